# Initial kernel scaffold; baseline (speedup 1.0000x reference)
import jax
import jax.numpy as jnp
from jax.experimental import pallas as pl


def kernel(hidden_states, gate_weight, gate_up_proj, down_proj, similarity_matrix):
    raise NotImplementedError("write your pallas kernel here")


# speedup vs baseline: 2.8821x; 2.8821x over previous
import jax
import jax.numpy as jnp
from jax.experimental import pallas as pl
from jax.experimental.pallas import tpu as pltpu

N_EXP = 8
D = 1024
DFF = 512
N_TOK = 2048
NEG = -3.0e38


def _argmax_lanes(v, iota_row):
    m = jnp.max(v, axis=-1, keepdims=True)
    return jnp.min(jnp.where(v == m, iota_row, N_EXP), axis=-1, keepdims=True), m


def _routing_body(x_ref, gw_ref, sim_ref, rw_ref):
    x = x_ref[...]
    gw = gw_ref[...]
    logits = jax.lax.dot_general(x, gw, (((1,), (1,)), ((), ())),
                                 preferred_element_type=jnp.float32)
    m = jnp.max(logits, axis=-1, keepdims=True)
    e = jnp.exp(logits - m)
    probs = e / jnp.sum(e, axis=-1, keepdims=True)

    iota_row = jax.lax.broadcasted_iota(jnp.int32, (N_TOK, N_EXP), 1)
    i1, v1 = _argmax_lanes(probs, iota_row)
    oh1 = (iota_row == i1)
    probs2 = jnp.where(oh1, NEG, probs)
    i2, v2 = _argmax_lanes(probs2, iota_row)
    oh2 = (iota_row == i2)
    denom = jnp.maximum(v1 + v2, 1e-12)
    w1 = v1 / denom
    w2 = v2 / denom

    mask_col = jnp.max(oh1.astype(jnp.float32), axis=0, keepdims=True)

    sim = sim_ref[...]
    iota_r8 = jax.lax.broadcasted_iota(jnp.int32, (N_EXP, N_EXP), 1)
    iota_c8 = jax.lax.broadcasted_iota(jnp.int32, (N_EXP, N_EXP), 0)
    eye = (iota_r8 == iota_c8)
    maskb = mask_col > 0.5
    sim_masked = jnp.where(maskb, sim, NEG)
    best_sim = jnp.max(sim_masked, axis=-1, keepdims=True)
    best_j = jnp.min(jnp.where(sim_masked == best_sim, iota_r8, N_EXP),
                     axis=-1, keepdims=True)
    mask_row = jnp.max(jnp.where(eye, jnp.broadcast_to(mask_col, (N_EXP, N_EXP)),
                                 0.0), axis=-1, keepdims=True)
    reroute = (mask_row < 0.5) & (best_sim >= 0.5)
    ident = jax.lax.broadcasted_iota(jnp.int32, (N_EXP, 1), 0)
    emap = jnp.where(reroute, best_j, ident)
    perm = (emap == iota_r8).astype(jnp.float32)

    pre = w1 * oh1.astype(jnp.float32) + w2 * oh2.astype(jnp.float32)
    rw = jax.lax.dot_general(pre, perm, (((1,), (0,)), ((), ())),
                             preferred_element_type=jnp.float32)
    rw_ref[...] = rw


def _ffn_body(x_ref, gup_ref, down_ref, w_ref, out_ref):
    e = pl.program_id(0)
    x = x_ref[...]
    gup = gup_ref[0]
    gu = jax.lax.dot_general(x, gup, (((1,), (1,)), ((), ())),
                             preferred_element_type=jnp.float32)
    gate = gu[:, :DFF]
    up = gu[:, DFF:]
    h = gate * jax.nn.sigmoid(gate) * up
    y = jax.lax.dot_general(h, down_ref[0], (((1,), (1,)), ((), ())),
                            preferred_element_type=jnp.float32)
    lanes = jax.lax.broadcasted_iota(jnp.int32, (N_TOK, N_EXP), 1)
    w_col = jnp.sum(jnp.where(lanes == e, w_ref[...], 0.0), axis=-1,
                    keepdims=True)
    y = y * w_col

    @pl.when(e == 0)
    def _():
        out_ref[...] = y

    @pl.when(e != 0)
    def _():
        out_ref[...] += y


def _routing(x, gate_weight, sim):
    return pl.pallas_call(
        _routing_body,
        out_shape=jax.ShapeDtypeStruct((N_TOK, N_EXP), jnp.float32),
        in_specs=[
            pl.BlockSpec((N_TOK, D), lambda: (0, 0)),
            pl.BlockSpec((N_EXP, D), lambda: (0, 0)),
            pl.BlockSpec((N_EXP, N_EXP), lambda: (0, 0)),
        ],
        out_specs=pl.BlockSpec((N_TOK, N_EXP), lambda: (0, 0)),
    )(x, gate_weight, sim)


def _ffn_dense(x, gate_up_proj, down_proj, rw):
    return pl.pallas_call(
        _ffn_body,
        grid=(N_EXP,),
        out_shape=jax.ShapeDtypeStruct((N_TOK, D), jnp.float32),
        in_specs=[
            pl.BlockSpec((N_TOK, D), lambda e: (0, 0)),
            pl.BlockSpec((1, 2 * DFF, D), lambda e: (e, 0, 0)),
            pl.BlockSpec((1, D, DFF), lambda e: (e, 0, 0)),
            pl.BlockSpec((N_TOK, N_EXP), lambda e: (0, 0)),
        ],
        out_specs=pl.BlockSpec((N_TOK, D), lambda e: (0, 0)),
    )(x, gate_up_proj, down_proj, rw)


def kernel(hidden_states, gate_weight, gate_up_proj, down_proj, similarity_matrix):
    B, S, Dm = hidden_states.shape
    x = hidden_states.reshape(-1, Dm)
    rw = _routing(x, gate_weight, similarity_matrix)
    out = _ffn_dense(x, gate_up_proj, down_proj, rw)
    return out.reshape(B, S, Dm)
